# baseline (speedup 1.0000x reference)
import functools

import jax
import jax.numpy as jnp
from jax import lax
from jax.experimental import pallas as pl
from jax.experimental.pallas import tpu as pltpu
from jax.experimental.pallas import tpu_sc as plsc

B = 16384
D = 100
DP = 128
NC = 2
NS = 16
NW = NC * NS
N_PER_W = B // NW
CHUNK = 256
NCHUNK = N_PER_W // CHUNK
NBUF = 3


def _make_sc_gather():
    mesh = plsc.VectorSubcoreMesh(
        core_axis_name="c", subcore_axis_name="s",
        num_cores=NC, num_subcores=NS)

    @functools.partial(
        pl.kernel,
        mesh=mesh,
        compiler_params=pltpu.CompilerParams(
            needs_layout_passes=False, use_tc_tiling_on_sc=False),
        out_type=[jax.ShapeDtypeStruct((B, DP), jnp.float32)] * 5,
        scratch_types=(
            [pltpu.VMEM((N_PER_W,), jnp.int32) for _ in range(5)]
            + [pltpu.VMEM((CHUNK, DP), jnp.float32) for _ in range(NBUF)]
            + [pltpu.SemaphoreType.DMA for _ in range(2 * NBUF + 1)]
        ),
    )
    def kern(gx_h, gy_h, day_h, hour_h, time_h,
             wgx_h, wgy_h, wday_h, whour_h, wtime_h,
             ogx, ogy, oday, ohour, otime,
             *scratch):
        idxs = scratch[:5]
        bufs = scratch[5:5 + NBUF]
        gsems = scratch[5 + NBUF:5 + 2 * NBUF]
        wsems = scratch[5 + 2 * NBUF:5 + 3 * NBUF]
        isem = scratch[5 + 3 * NBUF]
        wid = lax.axis_index("s") * NC + lax.axis_index("c")
        base = wid * N_PER_W
        streams = (
            (gx_h, wgx_h, ogx),
            (gy_h, wgy_h, ogy),
            (day_h, wday_h, oday),
            (hour_h, whour_h, ohour),
            (time_h, wtime_h, otime),
        )
        iloads = [
            pltpu.async_copy(streams[s][0].at[pl.ds(base, N_PER_W)],
                             idxs[s], isem)
            for s in range(5)
        ]
        for h in iloads:
            h.wait()
        for s, nrows in ((2, 7), (3, 24)):
            off = wid * nrows
            for k in range(N_PER_W // 16):
                idxs[s][pl.ds(16 * k, 16)] = idxs[s][pl.ds(16 * k, 16)] + off
        units = [(c, s) for c in range(NCHUNK) for s in range(5)]
        nu = len(units)
        LAG = 1
        gh = [None] * nu
        wh = [None] * nu
        for u in range(nu + LAG):
            if u < nu:
                c, s = units[u]
                b = u % NBUF
                if u >= NBUF:
                    wh[u - NBUF].wait()
                gh[u] = pltpu.async_copy(
                    streams[s][1].at[idxs[s].at[pl.ds(c * CHUNK, CHUNK)]],
                    bufs[b], gsems[b])
            if u >= LAG:
                up = u - LAG
                cp, sp = units[up]
                gh[up].wait()
                wh[up] = pltpu.async_copy(
                    bufs[up % NBUF],
                    streams[sp][2].at[pl.ds(base + cp * CHUNK, CHUNK)],
                    wsems[up % NBUF])
        for u in range(max(0, nu - NBUF), nu):
            wh[u].wait()

    return kern


_sc_gather = _make_sc_gather()

_TC_BLK = 1024


def _tc_concat_body(gx_ref, gy_ref, day_ref, hour_ref, time_ref,
                    sp_ref, tp_ref):
    sp_ref[...] = jnp.concatenate(
        [gx_ref[:, :D], gy_ref[:, :D]], axis=1)
    tp_ref[...] = jnp.concatenate(
        [day_ref[:, :D], hour_ref[:, :D], time_ref[:, :D]], axis=1)


_tc_concat = pl.pallas_call(
    _tc_concat_body,
    grid=(B // _TC_BLK,),
    in_specs=[pl.BlockSpec((_TC_BLK, DP), lambda i: (i, 0))] * 5,
    out_specs=[
        pl.BlockSpec((_TC_BLK, 2 * D), lambda i: (i, 0)),
        pl.BlockSpec((_TC_BLK, 3 * D), lambda i: (i, 0)),
    ],
    out_shape=[
        jax.ShapeDtypeStruct((B, 2 * D), jnp.float32),
        jax.ShapeDtypeStruct((B, 3 * D), jnp.float32),
    ],
)


def kernel(stats, day_bin, hour_bin, time_bin, G_X, G_Y,
           W_G_X, W_G_Y, W_day, W_hour, W_time):
    del stats
    pad = lambda w: jnp.pad(w, ((0, 0), (0, DP - D)))
    gxr, gyr, dayr, hourr, timer = _sc_gather(
        G_X.astype(jnp.int32), G_Y.astype(jnp.int32),
        day_bin.astype(jnp.int32), hour_bin.astype(jnp.int32),
        time_bin.astype(jnp.int32),
        pad(W_G_X), pad(W_G_Y),
        jnp.tile(pad(W_day), (NW, 1)), jnp.tile(pad(W_hour), (NW, 1)),
        pad(W_time),
    )
    return tuple(_tc_concat(gxr, gyr, dayr, hourr, timer))


# speedup vs baseline: 1.0550x; 1.0550x over previous
import functools

import jax
import jax.numpy as jnp
from jax import lax
from jax.experimental import pallas as pl
from jax.experimental.pallas import tpu as pltpu
from jax.experimental.pallas import tpu_sc as plsc

B = 16384
D = 100
DP = 128
NC = 2
NS = 16
NW = NC * NS
N_PER_W = B // NW
CHUNK = 256
NCHUNK = N_PER_W // CHUNK
NBUF = 3


def _make_sc_gather():
    mesh = plsc.VectorSubcoreMesh(
        core_axis_name="c", subcore_axis_name="s",
        num_cores=NC, num_subcores=NS)

    @functools.partial(
        pl.kernel,
        mesh=mesh,
        compiler_params=pltpu.CompilerParams(
            needs_layout_passes=False, use_tc_tiling_on_sc=False),
        out_type=[jax.ShapeDtypeStruct((B, DP), jnp.float32)] * 5,
        scratch_types=(
            [pltpu.VMEM((N_PER_W,), jnp.int32) for _ in range(5)]
            + [pltpu.VMEM((CHUNK, DP), jnp.float32) for _ in range(NBUF)]
            + [pltpu.SemaphoreType.DMA for _ in range(2 * NBUF + 1)]
        ),
    )
    def kern(gx_h, gy_h, day_h, hour_h, time_h,
             wgx_h, wgy_h, wday_h, whour_h, wtime_h,
             ogx, ogy, oday, ohour, otime,
             *scratch):
        idxs = scratch[:5]
        bufs = scratch[5:5 + NBUF]
        gsems = scratch[5 + NBUF:5 + 2 * NBUF]
        wsems = scratch[5 + 2 * NBUF:5 + 3 * NBUF]
        isem = scratch[5 + 3 * NBUF]
        wid = lax.axis_index("s") * NC + lax.axis_index("c")
        base = wid * N_PER_W
        streams = (
            (gx_h, wgx_h, ogx),
            (gy_h, wgy_h, ogy),
            (day_h, wday_h, oday),
            (hour_h, whour_h, ohour),
            (time_h, wtime_h, otime),
        )
        iloads = [
            pltpu.async_copy(streams[s][0].at[pl.ds(base, N_PER_W)],
                             idxs[s], isem)
            for s in range(5)
        ]
        for h in iloads:
            h.wait()
        for s, nrows, nrep in ((0, 256, 4), (1, 256, 4), (2, 7, 32),
                               (3, 24, 32), (4, 287, 8)):
            off = lax.rem(wid, nrep) * nrows
            for k in range(N_PER_W // 16):
                idxs[s][pl.ds(16 * k, 16)] = idxs[s][pl.ds(16 * k, 16)] + off
        units = [(c, s) for c in range(NCHUNK) for s in range(5)]
        nu = len(units)
        LAG = 1
        gh = [None] * nu
        wh = [None] * nu
        for u in range(nu + LAG):
            if u < nu:
                c, s = units[u]
                b = u % NBUF
                if u >= NBUF:
                    wh[u - NBUF].wait()
                gh[u] = pltpu.async_copy(
                    streams[s][1].at[idxs[s].at[pl.ds(c * CHUNK, CHUNK)]],
                    bufs[b], gsems[b])
            if u >= LAG:
                up = u - LAG
                cp, sp = units[up]
                gh[up].wait()
                wh[up] = pltpu.async_copy(
                    bufs[up % NBUF],
                    streams[sp][2].at[pl.ds(base + cp * CHUNK, CHUNK)],
                    wsems[up % NBUF])
        for u in range(max(0, nu - NBUF), nu):
            wh[u].wait()

    return kern


_sc_gather = _make_sc_gather()

_TC_BLK = 1024


def _tc_concat_body(gx_ref, gy_ref, day_ref, hour_ref, time_ref,
                    sp_ref, tp_ref):
    sp_ref[...] = jnp.concatenate(
        [gx_ref[:, :D], gy_ref[:, :D]], axis=1)
    tp_ref[...] = jnp.concatenate(
        [day_ref[:, :D], hour_ref[:, :D], time_ref[:, :D]], axis=1)


_tc_concat = pl.pallas_call(
    _tc_concat_body,
    grid=(B // _TC_BLK,),
    in_specs=[pl.BlockSpec((_TC_BLK, DP), lambda i: (i, 0))] * 5,
    out_specs=[
        pl.BlockSpec((_TC_BLK, 2 * D), lambda i: (i, 0)),
        pl.BlockSpec((_TC_BLK, 3 * D), lambda i: (i, 0)),
    ],
    out_shape=[
        jax.ShapeDtypeStruct((B, 2 * D), jnp.float32),
        jax.ShapeDtypeStruct((B, 3 * D), jnp.float32),
    ],
)


def kernel(stats, day_bin, hour_bin, time_bin, G_X, G_Y,
           W_G_X, W_G_Y, W_day, W_hour, W_time):
    del stats
    pad = lambda w: jnp.pad(w, ((0, 0), (0, DP - D)))
    gxr, gyr, dayr, hourr, timer = _sc_gather(
        G_X.astype(jnp.int32), G_Y.astype(jnp.int32),
        day_bin.astype(jnp.int32), hour_bin.astype(jnp.int32),
        time_bin.astype(jnp.int32),
        jnp.tile(pad(W_G_X), (4, 1)), jnp.tile(pad(W_G_Y), (4, 1)),
        jnp.tile(pad(W_day), (NW, 1)), jnp.tile(pad(W_hour), (NW, 1)),
        jnp.tile(pad(W_time), (8, 1)),
    )
    return tuple(_tc_concat(gxr, gyr, dayr, hourr, timer))


# speedup vs baseline: 1.0590x; 1.0038x over previous
import functools

import jax
import jax.numpy as jnp
from jax import lax
from jax.experimental import pallas as pl
from jax.experimental.pallas import tpu as pltpu
from jax.experimental.pallas import tpu_sc as plsc

B = 16384
D = 100
DP = 128
NC = 2
NS = 16
NW = NC * NS
N_PER_W = B // NW
CHUNK = 256
NCHUNK = N_PER_W // CHUNK
NBUF = 3


def _make_sc_gather():
    mesh = plsc.VectorSubcoreMesh(
        core_axis_name="c", subcore_axis_name="s",
        num_cores=NC, num_subcores=NS)

    @functools.partial(
        pl.kernel,
        mesh=mesh,
        compiler_params=pltpu.CompilerParams(
            needs_layout_passes=False, use_tc_tiling_on_sc=False),
        out_type=[jax.ShapeDtypeStruct((B, DP), jnp.float32)] * 5,
        scratch_types=(
            [pltpu.VMEM((N_PER_W,), jnp.int32) for _ in range(5)]
            + [pltpu.VMEM((CHUNK, DP), jnp.float32) for _ in range(NBUF)]
            + [pltpu.SemaphoreType.DMA for _ in range(2 * NBUF + 1)]
        ),
    )
    def kern(gx_h, gy_h, day_h, hour_h, time_h,
             wgx_h, wgy_h, wday_h, whour_h, wtime_h,
             ogx, ogy, oday, ohour, otime,
             *scratch):
        idxs = scratch[:5]
        bufs = scratch[5:5 + NBUF]
        gsems = scratch[5 + NBUF:5 + 2 * NBUF]
        wsems = scratch[5 + 2 * NBUF:5 + 3 * NBUF]
        isem = scratch[5 + 3 * NBUF]
        wid = lax.axis_index("s") * NC + lax.axis_index("c")
        base = wid * N_PER_W
        streams = (
            (gx_h, wgx_h, ogx),
            (gy_h, wgy_h, ogy),
            (day_h, wday_h, oday),
            (hour_h, whour_h, ohour),
            (time_h, wtime_h, otime),
        )
        iloads = [
            pltpu.async_copy(streams[s][0].at[pl.ds(base, N_PER_W)],
                             idxs[s], isem)
            for s in range(5)
        ]
        for h in iloads:
            h.wait()
        for s, nrows, nrep in ((0, 256, 8), (1, 256, 8), (2, 7, 32),
                               (3, 24, 32), (4, 287, 16)):
            off = lax.rem(wid, nrep) * nrows
            for k in range(N_PER_W // 16):
                idxs[s][pl.ds(16 * k, 16)] = idxs[s][pl.ds(16 * k, 16)] + off
        units = [(c, s) for c in range(NCHUNK) for s in range(5)]
        nu = len(units)
        LAG = 1
        gh = [None] * nu
        wh = [None] * nu
        for u in range(nu + LAG):
            if u < nu:
                c, s = units[u]
                b = u % NBUF
                if u >= NBUF:
                    wh[u - NBUF].wait()
                gh[u] = pltpu.async_copy(
                    streams[s][1].at[idxs[s].at[pl.ds(c * CHUNK, CHUNK)]],
                    bufs[b], gsems[b])
            if u >= LAG:
                up = u - LAG
                cp, sp = units[up]
                gh[up].wait()
                wh[up] = pltpu.async_copy(
                    bufs[up % NBUF],
                    streams[sp][2].at[pl.ds(base + cp * CHUNK, CHUNK)],
                    wsems[up % NBUF])
        for u in range(max(0, nu - NBUF), nu):
            wh[u].wait()

    return kern


_sc_gather = _make_sc_gather()

_TC_BLK = 1024


def _tc_concat_body(gx_ref, gy_ref, day_ref, hour_ref, time_ref,
                    sp_ref, tp_ref):
    sp_ref[...] = jnp.concatenate(
        [gx_ref[:, :D], gy_ref[:, :D]], axis=1)
    tp_ref[...] = jnp.concatenate(
        [day_ref[:, :D], hour_ref[:, :D], time_ref[:, :D]], axis=1)


_tc_concat = pl.pallas_call(
    _tc_concat_body,
    grid=(B // _TC_BLK,),
    in_specs=[pl.BlockSpec((_TC_BLK, DP), lambda i: (i, 0))] * 5,
    out_specs=[
        pl.BlockSpec((_TC_BLK, 2 * D), lambda i: (i, 0)),
        pl.BlockSpec((_TC_BLK, 3 * D), lambda i: (i, 0)),
    ],
    out_shape=[
        jax.ShapeDtypeStruct((B, 2 * D), jnp.float32),
        jax.ShapeDtypeStruct((B, 3 * D), jnp.float32),
    ],
)


def kernel(stats, day_bin, hour_bin, time_bin, G_X, G_Y,
           W_G_X, W_G_Y, W_day, W_hour, W_time):
    del stats
    pad = lambda w: jnp.pad(w, ((0, 0), (0, DP - D)))
    gxr, gyr, dayr, hourr, timer = _sc_gather(
        G_X.astype(jnp.int32), G_Y.astype(jnp.int32),
        day_bin.astype(jnp.int32), hour_bin.astype(jnp.int32),
        time_bin.astype(jnp.int32),
        jnp.tile(pad(W_G_X), (8, 1)), jnp.tile(pad(W_G_Y), (8, 1)),
        jnp.tile(pad(W_day), (NW, 1)), jnp.tile(pad(W_hour), (NW, 1)),
        jnp.tile(pad(W_time), (16, 1)),
    )
    return tuple(_tc_concat(gxr, gyr, dayr, hourr, timer))
